# baseline (speedup 1.0000x reference)
import functools

import jax
import jax.numpy as jnp
import numpy as np
from jax import lax
from jax.experimental import pallas as pl
from jax.experimental.pallas import tpu as pltpu
from jax.experimental.pallas import tpu_sc as plsc

NC = 2
NS = 16
L = 16
NW = NC * NS

W = 256
B = 512
SC_FRAC = 0.0


@functools.lru_cache(maxsize=None)
def _build_sc(T, N, A):
    C = A // NW
    NSLAB = C // W

    mesh = plsc.VectorSubcoreMesh(core_axis_name="c", subcore_axis_name="s")

    @functools.partial(
        pl.kernel,
        out_type=jax.ShapeDtypeStruct((A,), jnp.float32),
        mesh=mesh,
        compiler_params=pltpu.CompilerParams(needs_layout_passes=False),
        scratch_types=[
            pltpu.VMEM((C,), jnp.int32),
            pltpu.VMEM((C,), jnp.float32),
            pltpu.VMEM((C,), jnp.float32),
            pltpu.VMEM((2, T, W), jnp.float32),
            pltpu.VMEM((L,), jnp.int32),
            pltpu.SemaphoreType.DMA,
            pltpu.SemaphoreType.DMA,
        ],
    )
    def sc_kernel(buf_hbm, spikes_hbm, delays_hbm, ptr_hbm, out_hbm,
                  d_v, s_v, o_v, slab_v, p_v, sem0, sem1):
        wid = lax.axis_index("s") * NC + lax.axis_index("c")
        base = wid * C
        pltpu.sync_copy(delays_hbm.at[pl.ds(base, C)], d_v)
        pltpu.sync_copy(spikes_hbm.at[pl.ds(base, C)], s_v)
        pltpu.sync_copy(ptr_hbm, p_v)
        pvec = p_v[...]
        iot = lax.iota(jnp.int32, L)

        def fire(b, p, sem):
            pltpu.async_copy(
                buf_hbm.at[:, pl.ds(base + b * W, W)],
                slab_v.at[p], sem)

        def drain(p, sem):
            pltpu.make_async_copy(
                buf_hbm.at[:, pl.ds(base, W)], slab_v.at[p], sem).wait()

        def extract(b, p):
            slab = slab_v.at[p]
            for j in range(W // L):
                off = b * W + j * L
                d = d_v[pl.ds(off, L)]
                r = (pvec + T - d) & (T - 1)
                g = plsc.load_gather(slab, [r, j * L + iot])
                o_v[pl.ds(off, L)] = jnp.where(d == 0, s_v[pl.ds(off, L)], g)

        fire(0, 0, sem0)

        def body(k, _):
            b0 = 2 * k
            b1 = b0 + 1
            fire(b1, 1, sem1)
            drain(0, sem0)
            extract(b0, 0)

            @pl.when(b1 + 1 < NSLAB)
            def _():
                fire(b1 + 1, 0, sem0)

            drain(1, sem1)
            extract(b1, 1)
            return 0

        lax.fori_loop(0, NSLAB // 2, body, 0)
        pltpu.sync_copy(o_v, out_hbm.at[pl.ds(base, C)])

    return sc_kernel


@functools.lru_cache(maxsize=None)
def _build_tc(T, N, A):
    M = N - A
    g0 = A // B

    def tc_body(p2_ref, buf_ref, d_ref, s_ref, ptr_ref, o_ref):
        ptr = ptr_ref[0]
        d = d_ref[...]
        rr = (ptr + T - d) & (T - 1)
        packed = lax.dot_general(
            p2_ref[...], buf_ref[...].astype(jnp.bfloat16),
            (((1,), (0,)), ((), ())),
            preferred_element_type=jnp.float32)
        gi = rr >> 4
        m = lax.broadcasted_iota(jnp.int32, (8, B), 0) == gi
        val = jnp.sum(jnp.where(m, packed, 0.0), axis=0, keepdims=True)
        bit = (val.astype(jnp.int32) >> (rr & 15)) & 1
        o_ref[...] = jnp.where(d == 0, s_ref[...], bit.astype(jnp.float32))

    return pl.pallas_call(
        tc_body,
        grid=(M // B,),
        in_specs=[
            pl.BlockSpec((8, T), lambda g: (0, 0)),
            pl.BlockSpec((T, B), lambda g: (0, g0 + g)),
            pl.BlockSpec((1, B), lambda g: (0, g0 + g)),
            pl.BlockSpec((1, B), lambda g: (0, g0 + g)),
            pl.BlockSpec(memory_space=pltpu.SMEM),
        ],
        out_specs=pl.BlockSpec((1, B), lambda g: (0, g)),
        out_shape=jax.ShapeDtypeStruct((1, M), jnp.float32),
        compiler_params=pltpu.CompilerParams(
            dimension_semantics=("arbitrary",)),
    )


def _pack_matrix(T):
    t = np.arange(T)
    P = np.zeros((8, T), np.float32)
    P[t >> 4, t] = 2.0 ** (t & 15)
    return jnp.asarray(P, dtype=jnp.bfloat16)


def kernel(buffer, spikes, delays, ptr):
    T, N = buffer.shape
    A = int(N * SC_FRAC)
    A -= A % (NW * W)
    d32 = delays.astype(jnp.int32)
    s32 = spikes.astype(jnp.float32)
    ptr_s = jnp.reshape(jnp.asarray(ptr, jnp.int32), (1,))
    parts = []
    if A > 0:
        ptr_v = jnp.full((L,), ptr, dtype=jnp.int32)
        parts.append(_build_sc(T, N, A)(buffer, s32, d32, ptr_v))
    if A < N:
        out_tc = _build_tc(T, N, A)(
            _pack_matrix(T), buffer, d32.reshape(1, N), s32.reshape(1, N),
            ptr_s)
        parts.append(out_tc.reshape(N - A))
    return parts[0] if len(parts) == 1 else jnp.concatenate(parts)


# speedup vs baseline: 3.0197x; 3.0197x over previous
import functools

import jax
import jax.numpy as jnp
import numpy as np
from jax import lax
from jax.experimental import pallas as pl
from jax.experimental.pallas import tpu as pltpu
from jax.experimental.pallas import tpu_sc as plsc

NC = 2
NS = 16
L = 16
NW = NC * NS

W = 256
B = 2048
SC_FRAC = 0.0


@functools.lru_cache(maxsize=None)
def _build_sc(T, N, A):
    C = A // NW
    NSLAB = C // W

    mesh = plsc.VectorSubcoreMesh(core_axis_name="c", subcore_axis_name="s")

    @functools.partial(
        pl.kernel,
        out_type=jax.ShapeDtypeStruct((A,), jnp.float32),
        mesh=mesh,
        compiler_params=pltpu.CompilerParams(needs_layout_passes=False),
        scratch_types=[
            pltpu.VMEM((C,), jnp.int32),
            pltpu.VMEM((C,), jnp.float32),
            pltpu.VMEM((C,), jnp.float32),
            pltpu.VMEM((2, T, W), jnp.float32),
            pltpu.VMEM((L,), jnp.int32),
            pltpu.SemaphoreType.DMA,
            pltpu.SemaphoreType.DMA,
        ],
    )
    def sc_kernel(buf_hbm, spikes_hbm, delays_hbm, ptr_hbm, out_hbm,
                  d_v, s_v, o_v, slab_v, p_v, sem0, sem1):
        wid = lax.axis_index("s") * NC + lax.axis_index("c")
        base = wid * C
        pltpu.sync_copy(delays_hbm.at[pl.ds(base, C)], d_v)
        pltpu.sync_copy(spikes_hbm.at[pl.ds(base, C)], s_v)
        pltpu.sync_copy(ptr_hbm, p_v)
        pvec = p_v[...]
        iot = lax.iota(jnp.int32, L)

        def fire(b, p, sem):
            pltpu.async_copy(
                buf_hbm.at[:, pl.ds(base + b * W, W)],
                slab_v.at[p], sem)

        def drain(p, sem):
            pltpu.make_async_copy(
                buf_hbm.at[:, pl.ds(base, W)], slab_v.at[p], sem).wait()

        def extract(b, p):
            slab = slab_v.at[p]
            for j in range(W // L):
                off = b * W + j * L
                d = d_v[pl.ds(off, L)]
                r = (pvec + T - d) & (T - 1)
                g = plsc.load_gather(slab, [r, j * L + iot])
                o_v[pl.ds(off, L)] = jnp.where(d == 0, s_v[pl.ds(off, L)], g)

        fire(0, 0, sem0)

        def body(k, _):
            b0 = 2 * k
            b1 = b0 + 1
            fire(b1, 1, sem1)
            drain(0, sem0)
            extract(b0, 0)

            @pl.when(b1 + 1 < NSLAB)
            def _():
                fire(b1 + 1, 0, sem0)

            drain(1, sem1)
            extract(b1, 1)
            return 0

        lax.fori_loop(0, NSLAB // 2, body, 0)
        pltpu.sync_copy(o_v, out_hbm.at[pl.ds(base, C)])

    return sc_kernel


@functools.lru_cache(maxsize=None)
def _build_tc(T, N, A):
    M = N - A
    g0 = A // B

    def tc_body(p2_ref, buf_ref, d_ref, s_ref, ptr_ref, o_ref):
        ptr = ptr_ref[0]
        d = d_ref[...]
        rr = (ptr + T - d) & (T - 1)
        packed = lax.dot_general(
            p2_ref[...], buf_ref[...].astype(jnp.bfloat16),
            (((1,), (0,)), ((), ())),
            preferred_element_type=jnp.float32)
        gi = rr >> 4
        m = lax.broadcasted_iota(jnp.int32, (8, B), 0) == gi
        val = jnp.sum(jnp.where(m, packed, 0.0), axis=0, keepdims=True)
        bit = (val.astype(jnp.int32) >> (rr & 15)) & 1
        o_ref[...] = jnp.where(d == 0, s_ref[...], bit.astype(jnp.float32))

    return pl.pallas_call(
        tc_body,
        grid=(M // B,),
        in_specs=[
            pl.BlockSpec((8, T), lambda g: (0, 0)),
            pl.BlockSpec((T, B), lambda g: (0, g0 + g)),
            pl.BlockSpec((1, B), lambda g: (0, g0 + g)),
            pl.BlockSpec((1, B), lambda g: (0, g0 + g)),
            pl.BlockSpec(memory_space=pltpu.SMEM),
        ],
        out_specs=pl.BlockSpec((1, B), lambda g: (0, g)),
        out_shape=jax.ShapeDtypeStruct((1, M), jnp.float32),
        compiler_params=pltpu.CompilerParams(
            dimension_semantics=("arbitrary",)),
    )


def _pack_matrix(T):
    t = np.arange(T)
    P = np.zeros((8, T), np.float32)
    P[t >> 4, t] = 2.0 ** (t & 15)
    return jnp.asarray(P, dtype=jnp.bfloat16)


def kernel(buffer, spikes, delays, ptr):
    T, N = buffer.shape
    A = int(N * SC_FRAC)
    A -= A % (NW * W)
    d32 = delays.astype(jnp.int32)
    s32 = spikes.astype(jnp.float32)
    ptr_s = jnp.reshape(jnp.asarray(ptr, jnp.int32), (1,))
    parts = []
    if A > 0:
        ptr_v = jnp.full((L,), ptr, dtype=jnp.int32)
        parts.append(_build_sc(T, N, A)(buffer, s32, d32, ptr_v))
    if A < N:
        out_tc = _build_tc(T, N, A)(
            _pack_matrix(T), buffer, d32.reshape(1, N), s32.reshape(1, N),
            ptr_s)
        parts.append(out_tc.reshape(N - A))
    return parts[0] if len(parts) == 1 else jnp.concatenate(parts)


# speedup vs baseline: 4.5388x; 1.5031x over previous
import functools

import jax
import jax.numpy as jnp
import numpy as np
from jax import lax
from jax.experimental import pallas as pl
from jax.experimental.pallas import tpu as pltpu
from jax.experimental.pallas import tpu_sc as plsc

NC = 2
NS = 16
L = 16
NW = NC * NS

W = 256
B = 4096
SC_FRAC = 0.0


@functools.lru_cache(maxsize=None)
def _build_sc(T, N, A):
    C = A // NW
    NSLAB = C // W

    mesh = plsc.VectorSubcoreMesh(core_axis_name="c", subcore_axis_name="s")

    @functools.partial(
        pl.kernel,
        out_type=jax.ShapeDtypeStruct((A,), jnp.float32),
        mesh=mesh,
        compiler_params=pltpu.CompilerParams(needs_layout_passes=False),
        scratch_types=[
            pltpu.VMEM((C,), jnp.int32),
            pltpu.VMEM((C,), jnp.float32),
            pltpu.VMEM((C,), jnp.float32),
            pltpu.VMEM((2, T, W), jnp.float32),
            pltpu.VMEM((L,), jnp.int32),
            pltpu.SemaphoreType.DMA,
            pltpu.SemaphoreType.DMA,
        ],
    )
    def sc_kernel(buf_hbm, spikes_hbm, delays_hbm, ptr_hbm, out_hbm,
                  d_v, s_v, o_v, slab_v, p_v, sem0, sem1):
        wid = lax.axis_index("s") * NC + lax.axis_index("c")
        base = wid * C
        pltpu.sync_copy(delays_hbm.at[pl.ds(base, C)], d_v)
        pltpu.sync_copy(spikes_hbm.at[pl.ds(base, C)], s_v)
        pltpu.sync_copy(ptr_hbm, p_v)
        pvec = p_v[...]
        iot = lax.iota(jnp.int32, L)

        def fire(b, p, sem):
            pltpu.async_copy(
                buf_hbm.at[:, pl.ds(base + b * W, W)],
                slab_v.at[p], sem)

        def drain(p, sem):
            pltpu.make_async_copy(
                buf_hbm.at[:, pl.ds(base, W)], slab_v.at[p], sem).wait()

        def extract(b, p):
            slab = slab_v.at[p]
            for j in range(W // L):
                off = b * W + j * L
                d = d_v[pl.ds(off, L)]
                r = (pvec + T - d) & (T - 1)
                g = plsc.load_gather(slab, [r, j * L + iot])
                o_v[pl.ds(off, L)] = jnp.where(d == 0, s_v[pl.ds(off, L)], g)

        fire(0, 0, sem0)

        def body(k, _):
            b0 = 2 * k
            b1 = b0 + 1
            fire(b1, 1, sem1)
            drain(0, sem0)
            extract(b0, 0)

            @pl.when(b1 + 1 < NSLAB)
            def _():
                fire(b1 + 1, 0, sem0)

            drain(1, sem1)
            extract(b1, 1)
            return 0

        lax.fori_loop(0, NSLAB // 2, body, 0)
        pltpu.sync_copy(o_v, out_hbm.at[pl.ds(base, C)])

    return sc_kernel


@functools.lru_cache(maxsize=None)
def _build_tc(T, N, A):
    M = N - A
    g0 = A // B

    def tc_body(p2_ref, buf_ref, d_ref, s_ref, ptr_ref, o_ref):
        ptr = ptr_ref[0]
        d = d_ref[...]
        rr = (ptr + T - d) & (T - 1)
        packed = lax.dot_general(
            p2_ref[...], buf_ref[...].astype(jnp.bfloat16),
            (((1,), (0,)), ((), ())),
            preferred_element_type=jnp.float32)
        gi = rr >> 4
        m = lax.broadcasted_iota(jnp.int32, (8, B), 0) == gi
        val = jnp.sum(jnp.where(m, packed, 0.0), axis=0, keepdims=True)
        bit = (val.astype(jnp.int32) >> (rr & 15)) & 1
        o_ref[...] = jnp.where(d == 0, s_ref[...], bit.astype(jnp.float32))

    return pl.pallas_call(
        tc_body,
        grid=(M // B,),
        in_specs=[
            pl.BlockSpec((8, T), lambda g: (0, 0)),
            pl.BlockSpec((T, B), lambda g: (0, g0 + g)),
            pl.BlockSpec((1, B), lambda g: (0, g0 + g)),
            pl.BlockSpec((1, B), lambda g: (0, g0 + g)),
            pl.BlockSpec(memory_space=pltpu.SMEM),
        ],
        out_specs=pl.BlockSpec((1, B), lambda g: (0, g)),
        out_shape=jax.ShapeDtypeStruct((1, M), jnp.float32),
        compiler_params=pltpu.CompilerParams(
            dimension_semantics=("arbitrary",)),
    )


def _pack_matrix(T):
    t = np.arange(T)
    P = np.zeros((8, T), np.float32)
    P[t >> 4, t] = 2.0 ** (t & 15)
    return jnp.asarray(P, dtype=jnp.bfloat16)


def kernel(buffer, spikes, delays, ptr):
    T, N = buffer.shape
    A = int(N * SC_FRAC)
    A -= A % (NW * W)
    d32 = delays.astype(jnp.int32)
    s32 = spikes.astype(jnp.float32)
    ptr_s = jnp.reshape(jnp.asarray(ptr, jnp.int32), (1,))
    parts = []
    if A > 0:
        ptr_v = jnp.full((L,), ptr, dtype=jnp.int32)
        parts.append(_build_sc(T, N, A)(buffer, s32, d32, ptr_v))
    if A < N:
        out_tc = _build_tc(T, N, A)(
            _pack_matrix(T), buffer, d32.reshape(1, N), s32.reshape(1, N),
            ptr_s)
        parts.append(out_tc.reshape(N - A))
    return parts[0] if len(parts) == 1 else jnp.concatenate(parts)


# speedup vs baseline: 6.2493x; 1.3769x over previous
import functools

import jax
import jax.numpy as jnp
import numpy as np
from jax import lax
from jax.experimental import pallas as pl
from jax.experimental.pallas import tpu as pltpu
from jax.experimental.pallas import tpu_sc as plsc

NC = 2
NS = 16
L = 16
NW = NC * NS

W = 256
B = 8192
SC_FRAC = 0.0


@functools.lru_cache(maxsize=None)
def _build_sc(T, N, A):
    C = A // NW
    NSLAB = C // W

    mesh = plsc.VectorSubcoreMesh(core_axis_name="c", subcore_axis_name="s")

    @functools.partial(
        pl.kernel,
        out_type=jax.ShapeDtypeStruct((A,), jnp.float32),
        mesh=mesh,
        compiler_params=pltpu.CompilerParams(needs_layout_passes=False),
        scratch_types=[
            pltpu.VMEM((C,), jnp.int32),
            pltpu.VMEM((C,), jnp.float32),
            pltpu.VMEM((C,), jnp.float32),
            pltpu.VMEM((2, T, W), jnp.float32),
            pltpu.VMEM((L,), jnp.int32),
            pltpu.SemaphoreType.DMA,
            pltpu.SemaphoreType.DMA,
        ],
    )
    def sc_kernel(buf_hbm, spikes_hbm, delays_hbm, ptr_hbm, out_hbm,
                  d_v, s_v, o_v, slab_v, p_v, sem0, sem1):
        wid = lax.axis_index("s") * NC + lax.axis_index("c")
        base = wid * C
        pltpu.sync_copy(delays_hbm.at[pl.ds(base, C)], d_v)
        pltpu.sync_copy(spikes_hbm.at[pl.ds(base, C)], s_v)
        pltpu.sync_copy(ptr_hbm, p_v)
        pvec = p_v[...]
        iot = lax.iota(jnp.int32, L)

        def fire(b, p, sem):
            pltpu.async_copy(
                buf_hbm.at[:, pl.ds(base + b * W, W)],
                slab_v.at[p], sem)

        def drain(p, sem):
            pltpu.make_async_copy(
                buf_hbm.at[:, pl.ds(base, W)], slab_v.at[p], sem).wait()

        def extract(b, p):
            slab = slab_v.at[p]
            for j in range(W // L):
                off = b * W + j * L
                d = d_v[pl.ds(off, L)]
                r = (pvec + T - d) & (T - 1)
                g = plsc.load_gather(slab, [r, j * L + iot])
                o_v[pl.ds(off, L)] = jnp.where(d == 0, s_v[pl.ds(off, L)], g)

        fire(0, 0, sem0)

        def body(k, _):
            b0 = 2 * k
            b1 = b0 + 1
            fire(b1, 1, sem1)
            drain(0, sem0)
            extract(b0, 0)

            @pl.when(b1 + 1 < NSLAB)
            def _():
                fire(b1 + 1, 0, sem0)

            drain(1, sem1)
            extract(b1, 1)
            return 0

        lax.fori_loop(0, NSLAB // 2, body, 0)
        pltpu.sync_copy(o_v, out_hbm.at[pl.ds(base, C)])

    return sc_kernel


@functools.lru_cache(maxsize=None)
def _build_tc(T, N, A):
    M = N - A
    g0 = A // B

    def tc_body(p2_ref, buf_ref, d_ref, s_ref, ptr_ref, o_ref):
        ptr = ptr_ref[0]
        d = d_ref[...]
        rr = (ptr + T - d) & (T - 1)
        packed = lax.dot_general(
            p2_ref[...], buf_ref[...].astype(jnp.bfloat16),
            (((1,), (0,)), ((), ())),
            preferred_element_type=jnp.float32)
        gi = rr >> 4
        m = lax.broadcasted_iota(jnp.int32, (8, B), 0) == gi
        val = jnp.sum(jnp.where(m, packed, 0.0), axis=0, keepdims=True)
        bit = (val.astype(jnp.int32) >> (rr & 15)) & 1
        o_ref[...] = jnp.where(d == 0, s_ref[...], bit.astype(jnp.float32))

    return pl.pallas_call(
        tc_body,
        grid=(M // B,),
        in_specs=[
            pl.BlockSpec((8, T), lambda g: (0, 0)),
            pl.BlockSpec((T, B), lambda g: (0, g0 + g)),
            pl.BlockSpec((1, B), lambda g: (0, g0 + g)),
            pl.BlockSpec((1, B), lambda g: (0, g0 + g)),
            pl.BlockSpec(memory_space=pltpu.SMEM),
        ],
        out_specs=pl.BlockSpec((1, B), lambda g: (0, g)),
        out_shape=jax.ShapeDtypeStruct((1, M), jnp.float32),
        compiler_params=pltpu.CompilerParams(
            dimension_semantics=("arbitrary",)),
    )


def _pack_matrix(T):
    t = np.arange(T)
    P = np.zeros((8, T), np.float32)
    P[t >> 4, t] = 2.0 ** (t & 15)
    return jnp.asarray(P, dtype=jnp.bfloat16)


def kernel(buffer, spikes, delays, ptr):
    T, N = buffer.shape
    A = int(N * SC_FRAC)
    A -= A % (NW * W)
    d32 = delays.astype(jnp.int32)
    s32 = spikes.astype(jnp.float32)
    ptr_s = jnp.reshape(jnp.asarray(ptr, jnp.int32), (1,))
    parts = []
    if A > 0:
        ptr_v = jnp.full((L,), ptr, dtype=jnp.int32)
        parts.append(_build_sc(T, N, A)(buffer, s32, d32, ptr_v))
    if A < N:
        out_tc = _build_tc(T, N, A)(
            _pack_matrix(T), buffer, d32.reshape(1, N), s32.reshape(1, N),
            ptr_s)
        parts.append(out_tc.reshape(N - A))
    return parts[0] if len(parts) == 1 else jnp.concatenate(parts)


# speedup vs baseline: 6.8189x; 1.0911x over previous
import functools

import jax
import jax.numpy as jnp
import numpy as np
from jax import lax
from jax.experimental import pallas as pl
from jax.experimental.pallas import tpu as pltpu
from jax.experimental.pallas import tpu_sc as plsc

NC = 2
NS = 16
L = 16
NW = NC * NS

W = 256
B = 16384
SC_FRAC = 0.0


@functools.lru_cache(maxsize=None)
def _build_sc(T, N, A):
    C = A // NW
    NSLAB = C // W

    mesh = plsc.VectorSubcoreMesh(core_axis_name="c", subcore_axis_name="s")

    @functools.partial(
        pl.kernel,
        out_type=jax.ShapeDtypeStruct((A,), jnp.float32),
        mesh=mesh,
        compiler_params=pltpu.CompilerParams(needs_layout_passes=False),
        scratch_types=[
            pltpu.VMEM((C,), jnp.int32),
            pltpu.VMEM((C,), jnp.float32),
            pltpu.VMEM((C,), jnp.float32),
            pltpu.VMEM((2, T, W), jnp.float32),
            pltpu.VMEM((L,), jnp.int32),
            pltpu.SemaphoreType.DMA,
            pltpu.SemaphoreType.DMA,
        ],
    )
    def sc_kernel(buf_hbm, spikes_hbm, delays_hbm, ptr_hbm, out_hbm,
                  d_v, s_v, o_v, slab_v, p_v, sem0, sem1):
        wid = lax.axis_index("s") * NC + lax.axis_index("c")
        base = wid * C
        pltpu.sync_copy(delays_hbm.at[pl.ds(base, C)], d_v)
        pltpu.sync_copy(spikes_hbm.at[pl.ds(base, C)], s_v)
        pltpu.sync_copy(ptr_hbm, p_v)
        pvec = p_v[...]
        iot = lax.iota(jnp.int32, L)

        def fire(b, p, sem):
            pltpu.async_copy(
                buf_hbm.at[:, pl.ds(base + b * W, W)],
                slab_v.at[p], sem)

        def drain(p, sem):
            pltpu.make_async_copy(
                buf_hbm.at[:, pl.ds(base, W)], slab_v.at[p], sem).wait()

        def extract(b, p):
            slab = slab_v.at[p]
            for j in range(W // L):
                off = b * W + j * L
                d = d_v[pl.ds(off, L)]
                r = (pvec + T - d) & (T - 1)
                g = plsc.load_gather(slab, [r, j * L + iot])
                o_v[pl.ds(off, L)] = jnp.where(d == 0, s_v[pl.ds(off, L)], g)

        fire(0, 0, sem0)

        def body(k, _):
            b0 = 2 * k
            b1 = b0 + 1
            fire(b1, 1, sem1)
            drain(0, sem0)
            extract(b0, 0)

            @pl.when(b1 + 1 < NSLAB)
            def _():
                fire(b1 + 1, 0, sem0)

            drain(1, sem1)
            extract(b1, 1)
            return 0

        lax.fori_loop(0, NSLAB // 2, body, 0)
        pltpu.sync_copy(o_v, out_hbm.at[pl.ds(base, C)])

    return sc_kernel


@functools.lru_cache(maxsize=None)
def _build_tc(T, N, A):
    M = N - A
    g0 = A // B

    def tc_body(p2_ref, buf_ref, d_ref, s_ref, ptr_ref, o_ref):
        ptr = ptr_ref[0]
        d = d_ref[...]
        rr = (ptr + T - d) & (T - 1)
        packed = lax.dot_general(
            p2_ref[...], buf_ref[...].astype(jnp.bfloat16),
            (((1,), (0,)), ((), ())),
            preferred_element_type=jnp.float32)
        gi = rr >> 4
        m = lax.broadcasted_iota(jnp.int32, (8, B), 0) == gi
        val = jnp.sum(jnp.where(m, packed, 0.0), axis=0, keepdims=True)
        bit = (val.astype(jnp.int32) >> (rr & 15)) & 1
        o_ref[...] = jnp.where(d == 0, s_ref[...], bit.astype(jnp.float32))

    return pl.pallas_call(
        tc_body,
        grid=(M // B,),
        in_specs=[
            pl.BlockSpec((8, T), lambda g: (0, 0)),
            pl.BlockSpec((T, B), lambda g: (0, g0 + g)),
            pl.BlockSpec((1, B), lambda g: (0, g0 + g)),
            pl.BlockSpec((1, B), lambda g: (0, g0 + g)),
            pl.BlockSpec(memory_space=pltpu.SMEM),
        ],
        out_specs=pl.BlockSpec((1, B), lambda g: (0, g)),
        out_shape=jax.ShapeDtypeStruct((1, M), jnp.float32),
        compiler_params=pltpu.CompilerParams(
            dimension_semantics=("arbitrary",)),
    )


def _pack_matrix(T):
    t = np.arange(T)
    P = np.zeros((8, T), np.float32)
    P[t >> 4, t] = 2.0 ** (t & 15)
    return jnp.asarray(P, dtype=jnp.bfloat16)


def kernel(buffer, spikes, delays, ptr):
    T, N = buffer.shape
    A = int(N * SC_FRAC)
    A -= A % (NW * W)
    d32 = delays.astype(jnp.int32)
    s32 = spikes.astype(jnp.float32)
    ptr_s = jnp.reshape(jnp.asarray(ptr, jnp.int32), (1,))
    parts = []
    if A > 0:
        ptr_v = jnp.full((L,), ptr, dtype=jnp.int32)
        parts.append(_build_sc(T, N, A)(buffer, s32, d32, ptr_v))
    if A < N:
        out_tc = _build_tc(T, N, A)(
            _pack_matrix(T), buffer, d32.reshape(1, N), s32.reshape(1, N),
            ptr_s)
        parts.append(out_tc.reshape(N - A))
    return parts[0] if len(parts) == 1 else jnp.concatenate(parts)
